# Initial kernel scaffold; baseline (speedup 1.0000x reference)
import jax
import jax.numpy as jnp
from jax.experimental import pallas as pl


def kernel(x, edge_index, edge_weight, scalar0, scalar1, W, b):
    raise NotImplementedError("write your pallas kernel here")


# speedup vs baseline: 3.7810x; 3.7810x over previous
import functools

import jax
import jax.numpy as jnp
from jax import lax
from jax.experimental import pallas as pl
from jax.experimental.pallas import tpu as pltpu
from jax.experimental.pallas import tpu_sc as plsc

N = 10000
E = 320000
D = 128
NOUT = 128

NC = 2
NS = 16
NW = NC * NS
EPT = E // NW
K = 80
NCHUNK = EPT // K
RPT = N // NS

_mesh = plsc.VectorSubcoreMesh(
    core_axis_name="c", subcore_axis_name="s", num_cores=NC, num_subcores=NS
)


def _spmm_body(table, src, dst, w, out, acc, rows, srcv, dstv, wv, sem):
    c = lax.axis_index("c")
    s = lax.axis_index("s")
    wid = c * NS + s

    zero = jnp.zeros((16,), jnp.float32)

    def zrow(r, carry):
        for cix in range(8):
            rows[r, pl.ds(cix * 16, 16)] = zero
        return carry

    lax.fori_loop(0, K, zrow, 0)
    base_r = s * RPT
    for j in range(7):
        pltpu.sync_copy(rows, acc.at[pl.ds(base_r + j * K, K)])
    pltpu.sync_copy(rows.at[pl.ds(0, 65)], acc.at[pl.ds(base_r + 560, 65)])
    plsc.subcore_barrier()

    ebase = wid * EPT

    def chunk(i, carry):
        off = ebase + i * K
        pltpu.sync_copy(src.at[pl.ds(off, K)], srcv)
        pltpu.sync_copy(dst.at[pl.ds(off, K)], dstv)
        pltpu.sync_copy(w.at[pl.ds(off, K)], wv.at[pl.ds(0, K)])
        pltpu.async_copy(table.at[srcv], rows, sem).wait()

        def scale(e, inner):
            wvec = jnp.full((16,), wv[pl.ds(e, 16)][0], jnp.float32)
            for cix in range(8):
                sl = pl.ds(cix * 16, 16)
                rows[e, sl] = rows[e, sl] * wvec
            return inner

        lax.fori_loop(0, K, scale, 0)
        pltpu.sync_copy(rows, acc.at[dstv], add=True)
        return carry

    lax.fori_loop(0, NCHUNK, chunk, 0)
    plsc.subcore_barrier()

    start = pl.multiple_of(s * RPT - lax.rem(s, 8), 8)
    pltpu.sync_copy(
        acc.at[pl.ds(start, RPT + 7)],
        out.at[pl.ds(pl.multiple_of(c * N + start, 8), RPT + 7)],
    )


_spmm_sc = pl.kernel(
    _spmm_body,
    out_type=jax.ShapeDtypeStruct((NC * N, D), jnp.float32),
    mesh=_mesh,
    scratch_types=[
        pltpu.VMEM_SHARED((N, D), jnp.float32),
        pltpu.VMEM((K, D), jnp.float32),
        pltpu.VMEM((K,), jnp.int32),
        pltpu.VMEM((K,), jnp.int32),
        pltpu.VMEM((K + 16,), jnp.float32),
        pltpu.SemaphoreType.DMA,
    ],
)


def _elu(t):
    return jnp.where(t > 0, t, jnp.exp(jnp.minimum(t, 0.0)) - 1.0)


def _combine_body(p0, p1, o):
    o[...] = _elu(p0[...] + p1[...])


def _final_body(p0, p1, wt, bias, o):
    h = _elu(p0[...] + p1[...])
    o[...] = (
        lax.dot_general(
            h, wt[...], (((1,), (1,)), ((), ())),
            preferred_element_type=jnp.float32,
        )
        + bias[...]
    )


BR = 1000


def _combine(partials):
    return pl.pallas_call(
        _combine_body,
        grid=(N // BR,),
        in_specs=[
            pl.BlockSpec((BR, D), lambda i: (i, 0)),
            pl.BlockSpec((BR, D), lambda i: (i + N // BR, 0)),
        ],
        out_specs=pl.BlockSpec((BR, D), lambda i: (i, 0)),
        out_shape=jax.ShapeDtypeStruct((N, D), jnp.float32),
    )(partials, partials)


def _final(partials, W, b2):
    return pl.pallas_call(
        _final_body,
        grid=(N // BR,),
        in_specs=[
            pl.BlockSpec((BR, D), lambda i: (i, 0)),
            pl.BlockSpec((BR, D), lambda i: (i + N // BR, 0)),
            pl.BlockSpec((NOUT, D), lambda i: (0, 0)),
            pl.BlockSpec((1, NOUT), lambda i: (0, 0)),
        ],
        out_specs=pl.BlockSpec((BR, NOUT), lambda i: (i, 0)),
        out_shape=jax.ShapeDtypeStruct((N, NOUT), jnp.float32),
    )(partials, partials, W, b2)


@jax.jit
def kernel(x, edge_index, edge_weight, scalar0, scalar1, W, b):
    dst = edge_index[0]
    src = edge_index[1]
    w1 = edge_weight * scalar0[0]
    w2 = edge_weight * scalar1[0]
    p1 = _spmm_sc(x, src, dst, w1)
    h1 = _combine(p1)
    p2 = _spmm_sc(h1, src, dst, w2)
    return _final(p2, W, b.reshape(1, NOUT))


# speedup vs baseline: 9.6467x; 2.5513x over previous
import functools

import jax
import jax.numpy as jnp
from jax import lax
from jax.experimental import pallas as pl
from jax.experimental.pallas import tpu as pltpu
from jax.experimental.pallas import tpu_sc as plsc

N = 10000
E = 320000
D = 128
NOUT = 128

NC = 2
NS = 16
NW = NC * NS
EPT = E // NW
K = 80
NCHUNK = EPT // K
RPT = N // NS

_mesh = plsc.VectorSubcoreMesh(
    core_axis_name="c", subcore_axis_name="s", num_cores=NC, num_subcores=NS
)


def _gather_start(table, srcm, ci, buf, sem):
    pltpu.async_copy(table.at[srcm.at[ci]], buf, sem)


def _gather_wait(table, srcm, buf, sem):
    pltpu.make_async_copy(table.at[srcm.at[0]], buf, sem).wait()


def _scale_scatter(buf, wm, dstm, acc, ci):
    def scale(e, carry):
        wvec = jnp.full((16,), wm[ci, pl.ds(e, 16)][0], jnp.float32)
        for cix in range(8):
            sl = pl.ds(cix * 16, 16)
            buf[e, sl] = buf[e, sl] * wvec
        return carry

    lax.fori_loop(0, K, scale, 0, unroll=4)
    pltpu.sync_copy(buf, acc.at[dstm.at[ci]], add=True)


def _spmm_body(table, src3, dst3, w3, out, acc, bufa, bufb, srcm, dstm, wm,
               sema, semb):
    c = lax.axis_index("c")
    s = lax.axis_index("s")
    wid = c * NS + s

    pltpu.sync_copy(src3.at[wid], srcm)
    pltpu.sync_copy(dst3.at[wid], dstm)
    pltpu.sync_copy(w3.at[wid], wm.at[pl.ds(0, NCHUNK)])

    zero = jnp.zeros((16,), jnp.float32)

    def zrow(r, carry):
        for cix in range(8):
            bufa[r, pl.ds(cix * 16, 16)] = zero
        return carry

    lax.fori_loop(0, K, zrow, 0)
    base_r = s * RPT
    for j in range(7):
        pltpu.sync_copy(bufa, acc.at[pl.ds(base_r + j * K, K)])
    pltpu.sync_copy(bufa.at[pl.ds(0, 65)], acc.at[pl.ds(base_r + 560, 65)])
    plsc.subcore_barrier()

    _gather_start(table, srcm, 0, bufa, sema)

    def pair(j, carry):
        ca = 2 * j
        _gather_start(table, srcm, ca + 1, bufb, semb)
        _gather_wait(table, srcm, bufa, sema)
        _scale_scatter(bufa, wm, dstm, acc, ca)
        _gather_start(table, srcm, ca + 2, bufa, sema)
        _gather_wait(table, srcm, bufb, semb)
        _scale_scatter(bufb, wm, dstm, acc, ca + 1)
        return carry

    lax.fori_loop(0, (NCHUNK - 1) // 2, pair, 0)
    _gather_wait(table, srcm, bufa, sema)
    _scale_scatter(bufa, wm, dstm, acc, NCHUNK - 1)
    plsc.subcore_barrier()

    start = pl.multiple_of(s * RPT - lax.rem(s, 8), 8)
    pltpu.sync_copy(
        acc.at[pl.ds(start, RPT + 7)],
        out.at[pl.ds(pl.multiple_of(c * N + start, 8), RPT + 7)],
    )


_spmm_sc = pl.kernel(
    _spmm_body,
    out_type=jax.ShapeDtypeStruct((NC * N, D), jnp.float32),
    mesh=_mesh,
    scratch_types=[
        pltpu.VMEM_SHARED((N, D), jnp.float32),
        pltpu.VMEM((K, D), jnp.float32),
        pltpu.VMEM((K, D), jnp.float32),
        pltpu.VMEM((NCHUNK, K), jnp.int32),
        pltpu.VMEM((NCHUNK, K), jnp.int32),
        pltpu.VMEM((NCHUNK + 1, K), jnp.float32),
        pltpu.SemaphoreType.DMA,
        pltpu.SemaphoreType.DMA,
    ],
    compiler_params=pltpu.CompilerParams(use_tc_tiling_on_sc=False),
)


def _elu(t):
    return jnp.where(t > 0, t, jnp.exp(jnp.minimum(t, 0.0)) - 1.0)


def _combine_body(p0, p1, o):
    o[...] = _elu(p0[...] + p1[...])


def _final_body(p0, p1, wt, bias, o):
    h = _elu(p0[...] + p1[...])
    o[...] = (
        lax.dot_general(
            h, wt[...], (((1,), (1,)), ((), ())),
            preferred_element_type=jnp.float32,
        )
        + bias[...]
    )


BR = 1000


def _combine(partials):
    return pl.pallas_call(
        _combine_body,
        grid=(N // BR,),
        in_specs=[
            pl.BlockSpec((BR, D), lambda i: (i, 0)),
            pl.BlockSpec((BR, D), lambda i: (i + N // BR, 0)),
        ],
        out_specs=pl.BlockSpec((BR, D), lambda i: (i, 0)),
        out_shape=jax.ShapeDtypeStruct((N, D), jnp.float32),
    )(partials, partials)


def _final(partials, W, b2):
    return pl.pallas_call(
        _final_body,
        grid=(N // BR,),
        in_specs=[
            pl.BlockSpec((BR, D), lambda i: (i, 0)),
            pl.BlockSpec((BR, D), lambda i: (i + N // BR, 0)),
            pl.BlockSpec((NOUT, D), lambda i: (0, 0)),
            pl.BlockSpec((1, NOUT), lambda i: (0, 0)),
        ],
        out_specs=pl.BlockSpec((BR, NOUT), lambda i: (i, 0)),
        out_shape=jax.ShapeDtypeStruct((N, NOUT), jnp.float32),
    )(partials, partials, W, b2)


@jax.jit
def kernel(x, edge_index, edge_weight, scalar0, scalar1, W, b):
    dst = edge_index[0]
    src = edge_index[1]
    w1 = (edge_weight * scalar0[0]).reshape(NW, NCHUNK, K)
    w2 = (edge_weight * scalar1[0]).reshape(NW, NCHUNK, K)
    src3 = src.reshape(NW, NCHUNK, K)
    dst3 = dst.reshape(NW, NCHUNK, K)
    p1 = _spmm_sc(x, src3, dst3, w1)
    h1 = _combine(p1)
    p2 = _spmm_sc(h1, src3, dst3, w2)
    return _final(p2, W, b.reshape(1, NOUT))
